# baseline (speedup 1.0000x reference)
import functools

import jax
import jax.numpy as jnp
from jax import lax
from jax.experimental import pallas as pl
from jax.experimental.pallas import tpu as pltpu
from jax.experimental.pallas import tpu_sc as plsc

E = 8
D = 2048
T = 256
N = 2048
TB = 256
NB = N // TB + E
NPAD = NB * TB
CB = 256
RB = 256
PW = 128

_F32 = jnp.float32
_I32 = jnp.int32


def _router_body(x_ref, w_ref, b_ref, c_ref, idx_ref, prep_ref):
    x = x_ref[...].astype(jnp.bfloat16)
    z = jnp.dot(x, w_ref[...].astype(jnp.bfloat16),
                preferred_element_type=_F32) + b_ref[...]
    a = jax.nn.gelu(z)
    nrm = jnp.sqrt(jnp.sum(a * a, axis=-1, keepdims=True))
    dn = a / jnp.maximum(nrm, 1e-8)
    c = c_ref[...]
    cn = c / jnp.maximum(jnp.sqrt(jnp.sum(c * c, axis=-1, keepdims=True)), 1e-8)
    dc = lax.dot_general(dn.astype(jnp.bfloat16), cn.astype(jnp.bfloat16),
                         (((1,), (1,)), ((), ())),
                         preferred_element_type=_F32)
    sq = (jnp.sum(dn * dn, axis=-1, keepdims=True)
          + jnp.sum(cn * cn, axis=-1)[None, :] - 2.0 * dc)
    dist = jnp.sqrt(jnp.maximum(sq, 1e-12))
    neg = -dist
    mx = jnp.max(neg, axis=-1, keepdims=True)
    ex = jnp.exp(neg - mx)
    prob = ex / jnp.sum(ex, axis=-1, keepdims=True)
    p = jnp.max(prob, axis=-1)
    pm = jnp.max(prob, axis=-1, keepdims=True)
    io = lax.broadcasted_iota(_I32, (RB, E), 1)
    idx = jnp.min(jnp.where(prob >= pm, io, E), axis=-1)
    idx_ref[...] = idx.astype(_I32)
    prep_ref[...] = jnp.broadcast_to(p[:, None], (RB, PW))


def _router(premise, W_dist, b_dist2d, centroids):
    return pl.pallas_call(
        _router_body,
        grid=(N // RB,),
        in_specs=[
            pl.BlockSpec((RB, D), lambda i: (i, 0)),
            pl.BlockSpec((D, T), lambda i: (0, 0)),
            pl.BlockSpec((1, T), lambda i: (0, 0)),
            pl.BlockSpec((E, T), lambda i: (0, 0)),
        ],
        out_specs=[
            pl.BlockSpec((RB,), lambda i: (i,)),
            pl.BlockSpec((RB, PW), lambda i: (i, 0)),
        ],
        out_shape=[
            jax.ShapeDtypeStruct((N,), _I32),
            jax.ShapeDtypeStruct((N, PW), _F32),
        ],
    )(premise, W_dist, b_dist2d, centroids)


def _dispatch_body(idx_ref, pos_ref, eid_ref, rank_scr):
    nchunks = N // CB
    tri = (lax.broadcasted_iota(_I32, (CB, CB), 0)
           > lax.broadcasted_iota(_I32, (CB, CB), 1)).astype(_F32)
    ioe = lax.broadcasted_iota(_I32, (1, E), 1)
    base = jnp.zeros((1, E), _F32)
    for c in range(nchunks):
        iv = idx_ref[pl.ds(c * CB, CB)]
        mc = (iv[:, None] == ioe).astype(_F32)
        excl = jnp.dot(tri, mc, preferred_element_type=_F32) + base
        rank_scr[c, :] = jnp.sum(excl * mc, axis=1)
        base = base + jnp.sum(mc, axis=0, keepdims=True)
    counts = base
    nblk = jnp.floor((counts + (TB - 1)) / TB)
    triu = (lax.broadcasted_iota(_I32, (E, E), 0)
            <= lax.broadcasted_iota(_I32, (E, E), 1)).astype(_F32)
    endblk = jnp.dot(nblk, triu, preferred_element_type=_F32)
    sb = endblk - nblk
    bst = sb * TB
    total = endblk[:, E - 1:E]
    last_eid = jnp.sum((endblk <= total - 1.0).astype(_F32),
                       axis=1, keepdims=True)
    bio = lax.broadcasted_iota(_I32, (NB, E), 0).astype(_F32)
    raw = jnp.sum((jnp.broadcast_to(endblk, (NB, E)) <= bio).astype(_F32),
                  axis=1, keepdims=True)
    eid = jnp.minimum(raw, last_eid)
    xmap = jnp.minimum(bio[:, :1], jnp.broadcast_to(total - 1.0, (NB, 1)))
    eid_ref[...] = jnp.concatenate([eid, xmap], axis=1).astype(_I32)
    for c in range(nchunks):
        iv = idx_ref[pl.ds(c * CB, CB)]
        mc = (iv[:, None] == ioe).astype(_F32)
        start = jnp.sum(mc * bst, axis=1)
        pos_ref[pl.ds(c * CB, CB)] = (start + rank_scr[c, :]).astype(_I32)


def _dispatch(idx):
    return pl.pallas_call(
        _dispatch_body,
        in_specs=[pl.BlockSpec((N,), lambda: (0,))],
        out_specs=[
            pl.BlockSpec((N,), lambda: (0,)),
            pl.BlockSpec((NB, 2), lambda: (0, 0)),
        ],
        out_shape=[
            jax.ShapeDtypeStruct((N,), _I32),
            jax.ShapeDtypeStruct((NB, 2), _I32),
        ],
        scratch_shapes=[pltpu.VMEM((N // CB, CB), _F32)],
    )(idx)


_NC = 2
_NS = 16
_NW = _NC * _NS
_TPW = N // _NW
_CH = 16
_NCH = _TPW // _CH


def _sc_mesh():
    return plsc.VectorSubcoreMesh(core_axis_name="c", subcore_axis_name="s",
                                  num_cores=_NC, num_subcores=_NS)


def _sc_scatter_body(prem_hbm, prep_hbm, pos_hbm, sx_hbm, sp_hbm,
                     pos_v, rows_v, pr_v, rsem, wsem):
    wid = lax.axis_index("s") * _NC + lax.axis_index("c")
    base = wid * _TPW

    def rd(c):
        cb = base + c * _CH
        k = c % 2
        return (
            pltpu.async_copy(pos_hbm.at[pl.ds(cb, _CH)], pos_v[k], rsem[k]),
            pltpu.async_copy(prem_hbm.at[pl.ds(cb, _CH)], rows_v[k], rsem[k]),
            pltpu.async_copy(prep_hbm.at[pl.ds(cb, _CH)], pr_v[k], rsem[k]),
        )

    def wt(c):
        k = c % 2
        return (
            pltpu.async_copy(rows_v[k], sx_hbm.at[pos_v[k]], wsem[k]),
            pltpu.async_copy(pr_v[k], sp_hbm.at[pos_v[k]], wsem[k]),
        )

    reads = {0: rd(0), 1: rd(1)}
    writes = {}
    for c in range(_NCH):
        for cp in reads[c]:
            cp.wait()
        writes[c] = wt(c)
        if c + 2 < _NCH:
            for cp in writes[c]:
                cp.wait()
            reads[c + 2] = rd(c + 2)
    for c in (_NCH - 2, _NCH - 1):
        for cp in writes[c]:
            cp.wait()


@functools.cache
def _sc_scatter_kernel():
    return pl.kernel(
        _sc_scatter_body,
        out_type=(jax.ShapeDtypeStruct((NPAD, D), _F32),
                  jax.ShapeDtypeStruct((NPAD, PW), _F32)),
        mesh=_sc_mesh(),
        scratch_types=[
            [pltpu.VMEM((_CH,), _I32) for _ in range(2)],
            [pltpu.VMEM((_CH, D), _F32) for _ in range(2)],
            [pltpu.VMEM((_CH, PW), _F32) for _ in range(2)],
            [pltpu.SemaphoreType.DMA for _ in range(2)],
            [pltpu.SemaphoreType.DMA for _ in range(2)],
        ],
    )


NF = 2
FB = D // NF


DH = D // 2


def _expert_body(eid_ref, x_ref, p_ref, wh0_ref, wh1_ref, bh_ref,
                 wg0_ref, wg1_ref, bg_ref, o_ref):
    j = pl.program_id(0)
    b = pl.program_id(1)
    x = x_ref[...]
    xb = x.astype(jnp.bfloat16)
    x0, x1 = xb[:, :DH], xb[:, DH:]
    ah = (jnp.dot(x0, wh0_ref[0].astype(jnp.bfloat16),
                  preferred_element_type=_F32)
          + jnp.dot(x1, wh1_ref[0].astype(jnp.bfloat16),
                    preferred_element_type=_F32) + bh_ref[0])
    h = jax.nn.gelu(ah)
    ag = (jnp.dot(x0, wg0_ref[0].astype(jnp.bfloat16),
                  preferred_element_type=_F32)
          + jnp.dot(x1, wg1_ref[0].astype(jnp.bfloat16),
                    preferred_element_type=_F32) + bg_ref[0])
    g = jax.nn.sigmoid(ag)
    p = p_ref[...][:, :1]
    xs = jnp.where(jnp.broadcast_to(j == 0, (TB, FB)), x[:, :FB], x[:, FB:])

    @pl.when(eid_ref[b, 1] == b)
    def _store():
        o_ref[...] = (g * h + (1.0 - g) * xs) * p


def _experts(eid, sx, sp, Wh, bh3, Wg, bg3):
    wspec0 = pl.BlockSpec((1, DH, FB), lambda j, b, eid: (eid[b, 0], 0, j))
    wspec1 = pl.BlockSpec((1, DH, FB), lambda j, b, eid: (eid[b, 0], 1, j))
    return pl.pallas_call(
        _expert_body,
        grid_spec=pltpu.PrefetchScalarGridSpec(
            num_scalar_prefetch=1,
            grid=(NF, NB),
            in_specs=[
                pl.BlockSpec((TB, D), lambda j, b, eid: (eid[b, 1], 0)),
                pl.BlockSpec((TB, PW), lambda j, b, eid: (eid[b, 1], 0)),
                wspec0, wspec1,
                pl.BlockSpec((1, 1, FB), lambda j, b, eid: (eid[b, 0], 0, j)),
                wspec0, wspec1,
                pl.BlockSpec((1, 1, FB), lambda j, b, eid: (eid[b, 0], 0, j)),
            ],
            out_specs=pl.BlockSpec((TB, FB), lambda j, b, eid: (eid[b, 1], j)),
        ),
        out_shape=jax.ShapeDtypeStruct((NPAD, D), _F32),
    )(eid, sx, sp, Wh, Wh, bh3, Wg, Wg, bg3)


def _sc_gather_body(res_hbm, pos_hbm, out_hbm, pos_v, rows_v, gsem, wsem):
    wid = lax.axis_index("s") * _NC + lax.axis_index("c")
    base = wid * _TPW

    psems = [pltpu.async_copy(pos_hbm.at[pl.ds(base + c * _CH, _CH)],
                              pos_v[c], gsem[c % 2])
             for c in range(_NCH)]
    for cp in psems:
        cp.wait()

    def gt(c):
        k = c % 2
        return pltpu.async_copy(res_hbm.at[pos_v[c]], rows_v[k], gsem[k])

    def wt(c):
        k = c % 2
        cb = base + c * _CH
        return pltpu.async_copy(rows_v[k], out_hbm.at[pl.ds(cb, _CH)],
                                wsem[k])

    gets = {0: gt(0), 1: gt(1)}
    writes = {}
    for c in range(_NCH):
        gets[c].wait()
        writes[c] = wt(c)
        if c + 2 < _NCH:
            writes[c].wait()
            gets[c + 2] = gt(c + 2)
    for c in (_NCH - 2, _NCH - 1):
        writes[c].wait()


@functools.cache
def _sc_gather_kernel():
    return pl.kernel(
        _sc_gather_body,
        out_type=jax.ShapeDtypeStruct((N, D), _F32),
        mesh=_sc_mesh(),
        scratch_types=[
            [pltpu.VMEM((_CH,), _I32) for _ in range(_NCH)],
            [pltpu.VMEM((_CH, D), _F32) for _ in range(2)],
            [pltpu.SemaphoreType.DMA for _ in range(2)],
            [pltpu.SemaphoreType.DMA for _ in range(2)],
        ],
    )


def kernel(premise, centroids, W_dist, b_dist, Wh, bh, Wg, bg):
    idx, prep = _router(premise, W_dist, b_dist.reshape(1, T), centroids)
    pos, eid = _dispatch(idx)
    sx, sp = _sc_scatter_kernel()(premise, prep, pos)
    res = _experts(eid, sx, sp, Wh, bh.reshape(E, 1, D), Wg, bg.reshape(E, 1, D))
    return _sc_gather_kernel()(res, pos)


# speedup vs baseline: 1.0084x; 1.0084x over previous
import functools

import jax
import jax.numpy as jnp
from jax import lax
from jax.experimental import pallas as pl
from jax.experimental.pallas import tpu as pltpu
from jax.experimental.pallas import tpu_sc as plsc

E = 8
D = 2048
T = 256
N = 2048
TB = 256
NB = N // TB + E
NPAD = NB * TB
CB = 256
RB = 256
PW = 128

_F32 = jnp.float32
_I32 = jnp.int32


def _router_body(x_ref, w_ref, b_ref, c_ref, idx_ref, prep_ref):
    x = x_ref[...].astype(jnp.bfloat16)
    z = jnp.dot(x, w_ref[...].astype(jnp.bfloat16),
                preferred_element_type=_F32) + b_ref[...]
    a = jax.nn.gelu(z)
    nrm = jnp.sqrt(jnp.sum(a * a, axis=-1, keepdims=True))
    dn = a / jnp.maximum(nrm, 1e-8)
    c = c_ref[...]
    cn = c / jnp.maximum(jnp.sqrt(jnp.sum(c * c, axis=-1, keepdims=True)), 1e-8)
    dc = lax.dot_general(dn.astype(jnp.bfloat16), cn.astype(jnp.bfloat16),
                         (((1,), (1,)), ((), ())),
                         preferred_element_type=_F32)
    sq = (jnp.sum(dn * dn, axis=-1, keepdims=True)
          + jnp.sum(cn * cn, axis=-1)[None, :] - 2.0 * dc)
    dist = jnp.sqrt(jnp.maximum(sq, 1e-12))
    neg = -dist
    mx = jnp.max(neg, axis=-1, keepdims=True)
    ex = jnp.exp(neg - mx)
    prob = ex / jnp.sum(ex, axis=-1, keepdims=True)
    p = jnp.max(prob, axis=-1)
    pm = jnp.max(prob, axis=-1, keepdims=True)
    io = lax.broadcasted_iota(_I32, (RB, E), 1)
    idx = jnp.min(jnp.where(prob >= pm, io, E), axis=-1)
    idx_ref[...] = idx.astype(_I32)
    prep_ref[...] = jnp.broadcast_to(p[:, None], (RB, PW))


def _router(premise, W_dist, b_dist2d, centroids):
    return pl.pallas_call(
        _router_body,
        grid=(N // RB,),
        in_specs=[
            pl.BlockSpec((RB, D), lambda i: (i, 0)),
            pl.BlockSpec((D, T), lambda i: (0, 0)),
            pl.BlockSpec((1, T), lambda i: (0, 0)),
            pl.BlockSpec((E, T), lambda i: (0, 0)),
        ],
        out_specs=[
            pl.BlockSpec((RB,), lambda i: (i,)),
            pl.BlockSpec((RB, PW), lambda i: (i, 0)),
        ],
        out_shape=[
            jax.ShapeDtypeStruct((N,), _I32),
            jax.ShapeDtypeStruct((N, PW), _F32),
        ],
    )(premise, W_dist, b_dist2d, centroids)


def _dispatch_body(idx_ref, pos_ref, eid_ref, rank_scr):
    nchunks = N // CB
    tri = (lax.broadcasted_iota(_I32, (CB, CB), 0)
           > lax.broadcasted_iota(_I32, (CB, CB), 1)).astype(_F32)
    ioe = lax.broadcasted_iota(_I32, (1, E), 1)
    base = jnp.zeros((1, E), _F32)
    for c in range(nchunks):
        iv = idx_ref[pl.ds(c * CB, CB)]
        mc = (iv[:, None] == ioe).astype(_F32)
        excl = jnp.dot(tri, mc, preferred_element_type=_F32) + base
        rank_scr[c, :] = jnp.sum(excl * mc, axis=1)
        base = base + jnp.sum(mc, axis=0, keepdims=True)
    counts = base
    nblk = jnp.floor((counts + (TB - 1)) / TB)
    triu = (lax.broadcasted_iota(_I32, (E, E), 0)
            <= lax.broadcasted_iota(_I32, (E, E), 1)).astype(_F32)
    endblk = jnp.dot(nblk, triu, preferred_element_type=_F32)
    sb = endblk - nblk
    bst = sb * TB
    total = endblk[:, E - 1:E]
    last_eid = jnp.sum((endblk <= total - 1.0).astype(_F32),
                       axis=1, keepdims=True)
    bio = lax.broadcasted_iota(_I32, (NB, E), 0).astype(_F32)
    raw = jnp.sum((jnp.broadcast_to(endblk, (NB, E)) <= bio).astype(_F32),
                  axis=1, keepdims=True)
    eid = jnp.minimum(raw, last_eid)
    xmap = jnp.minimum(bio[:, :1], jnp.broadcast_to(total - 1.0, (NB, 1)))
    eid_ref[...] = jnp.concatenate([eid, xmap], axis=1).astype(_I32)
    for c in range(nchunks):
        iv = idx_ref[pl.ds(c * CB, CB)]
        mc = (iv[:, None] == ioe).astype(_F32)
        start = jnp.sum(mc * bst, axis=1)
        pos_ref[pl.ds(c * CB, CB)] = (start + rank_scr[c, :]).astype(_I32)


def _dispatch(idx):
    return pl.pallas_call(
        _dispatch_body,
        in_specs=[pl.BlockSpec((N,), lambda: (0,))],
        out_specs=[
            pl.BlockSpec((N,), lambda: (0,)),
            pl.BlockSpec((NB, 2), lambda: (0, 0)),
        ],
        out_shape=[
            jax.ShapeDtypeStruct((N,), _I32),
            jax.ShapeDtypeStruct((NB, 2), _I32),
        ],
        scratch_shapes=[pltpu.VMEM((N // CB, CB), _F32)],
    )(idx)


_NC = 2
_NS = 16
_NW = _NC * _NS
_TPW = N // _NW
_CH = 16
_NCH = _TPW // _CH


def _sc_mesh():
    return plsc.VectorSubcoreMesh(core_axis_name="c", subcore_axis_name="s",
                                  num_cores=_NC, num_subcores=_NS)


def _sc_scatter_body(prem_hbm, prep_hbm, pos_hbm, sx_hbm, sp_hbm,
                     pos_v, rows_v, pr_v, rsem, wsem):
    wid = lax.axis_index("s") * _NC + lax.axis_index("c")
    base = wid * _TPW

    def rd(c):
        cb = base + c * _CH
        k = c % 2
        return (
            pltpu.async_copy(pos_hbm.at[pl.ds(cb, _CH)], pos_v[k], rsem[k]),
            pltpu.async_copy(prem_hbm.at[pl.ds(cb, _CH)], rows_v[k], rsem[k]),
            pltpu.async_copy(prep_hbm.at[pl.ds(cb, _CH)], pr_v[k], rsem[k]),
        )

    def wt(c):
        k = c % 2
        return (
            pltpu.async_copy(rows_v[k], sx_hbm.at[pos_v[k]], wsem[k]),
            pltpu.async_copy(pr_v[k], sp_hbm.at[pos_v[k]], wsem[k]),
        )

    reads = {0: rd(0), 1: rd(1)}
    writes = {}
    for c in range(_NCH):
        for cp in reads[c]:
            cp.wait()
        writes[c] = wt(c)
        if c + 2 < _NCH:
            for cp in writes[c]:
                cp.wait()
            reads[c + 2] = rd(c + 2)
    for c in (_NCH - 2, _NCH - 1):
        for cp in writes[c]:
            cp.wait()


@functools.cache
def _sc_scatter_kernel():
    return pl.kernel(
        _sc_scatter_body,
        out_type=(jax.ShapeDtypeStruct((NPAD, D), _F32),
                  jax.ShapeDtypeStruct((NPAD, PW), _F32)),
        mesh=_sc_mesh(),
        scratch_types=[
            [pltpu.VMEM((_CH,), _I32) for _ in range(2)],
            [pltpu.VMEM((_CH, D), _F32) for _ in range(2)],
            [pltpu.VMEM((_CH, PW), _F32) for _ in range(2)],
            [pltpu.SemaphoreType.DMA for _ in range(2)],
            [pltpu.SemaphoreType.DMA for _ in range(2)],
        ],
    )


NF = 2
FB = D // NF


def _expert_body(eid_ref, x_ref, p_ref, wh_ref, bh_ref, wg_ref, bg_ref,
                 o_ref):
    j = pl.program_id(0)
    b = pl.program_id(1)
    x = x_ref[...]
    xb = x.astype(jnp.bfloat16)
    wh = wh_ref[0].astype(jnp.bfloat16)
    wg = wg_ref[0].astype(jnp.bfloat16)
    ah = jnp.dot(xb, wh, preferred_element_type=_F32) + bh_ref[0]
    h = jax.nn.gelu(ah)
    ag = jnp.dot(xb, wg, preferred_element_type=_F32) + bg_ref[0]
    g = jax.nn.sigmoid(ag)
    p = p_ref[...][:, :1]
    xs = jnp.where(jnp.broadcast_to(j == 0, (TB, FB)), x[:, :FB], x[:, FB:])

    @pl.when(eid_ref[b, 1] == b)
    def _store():
        o_ref[...] = (g * h + (1.0 - g) * xs) * p


def _experts(eid, sx, sp, Wh, bh3, Wg, bg3):
    return pl.pallas_call(
        _expert_body,
        grid_spec=pltpu.PrefetchScalarGridSpec(
            num_scalar_prefetch=1,
            grid=(NF, NB),
            in_specs=[
                pl.BlockSpec((TB, D), lambda j, b, eid: (eid[b, 1], 0)),
                pl.BlockSpec((TB, PW), lambda j, b, eid: (eid[b, 1], 0)),
                pl.BlockSpec((1, D, FB), lambda j, b, eid: (eid[b, 0], 0, j)),
                pl.BlockSpec((1, 1, FB), lambda j, b, eid: (eid[b, 0], 0, j)),
                pl.BlockSpec((1, D, FB), lambda j, b, eid: (eid[b, 0], 0, j)),
                pl.BlockSpec((1, 1, FB), lambda j, b, eid: (eid[b, 0], 0, j)),
            ],
            out_specs=pl.BlockSpec((TB, FB), lambda j, b, eid: (eid[b, 1], j)),
        ),
        out_shape=jax.ShapeDtypeStruct((NPAD, D), _F32),
    )(eid, sx, sp, Wh, bh3, Wg, bg3)


def _sc_gather_body(res_hbm, pos_hbm, out_hbm, pos_v, rows_v, gsem, wsem):
    wid = lax.axis_index("s") * _NC + lax.axis_index("c")
    base = wid * _TPW

    psems = [pltpu.async_copy(pos_hbm.at[pl.ds(base + c * _CH, _CH)],
                              pos_v[c], gsem[c % 2])
             for c in range(_NCH)]
    for cp in psems:
        cp.wait()

    def gt(c):
        k = c % 2
        return pltpu.async_copy(res_hbm.at[pos_v[c]], rows_v[k], gsem[k])

    def wt(c):
        k = c % 2
        cb = base + c * _CH
        return pltpu.async_copy(rows_v[k], out_hbm.at[pl.ds(cb, _CH)],
                                wsem[k])

    gets = {0: gt(0), 1: gt(1)}
    writes = {}
    for c in range(_NCH):
        gets[c].wait()
        writes[c] = wt(c)
        if c + 2 < _NCH:
            writes[c].wait()
            gets[c + 2] = gt(c + 2)
    for c in (_NCH - 2, _NCH - 1):
        writes[c].wait()


@functools.cache
def _sc_gather_kernel():
    return pl.kernel(
        _sc_gather_body,
        out_type=jax.ShapeDtypeStruct((N, D), _F32),
        mesh=_sc_mesh(),
        scratch_types=[
            [pltpu.VMEM((_CH,), _I32) for _ in range(_NCH)],
            [pltpu.VMEM((_CH, D), _F32) for _ in range(2)],
            [pltpu.SemaphoreType.DMA for _ in range(2)],
            [pltpu.SemaphoreType.DMA for _ in range(2)],
        ],
    )


def kernel(premise, centroids, W_dist, b_dist, Wh, bh, Wg, bg):
    idx, prep = _router(premise, W_dist, b_dist.reshape(1, T), centroids)
    pos, eid = _dispatch(idx)
    sx, sp = _sc_scatter_kernel()(premise, prep, pos)
    res = _experts(eid, sx, sp, Wh, bh.reshape(E, 1, D), Wg, bg.reshape(E, 1, D))
    return _sc_gather_kernel()(res, pos)
